# baseline (speedup 1.0000x reference)
import functools

import jax
import jax.numpy as jnp
from jax import lax
from jax.experimental import pallas as pl
from jax.experimental.pallas import tpu as pltpu
from jax.experimental.pallas import tpu_sc as plsc

D = 64
CHUNK = 128
NC = 2
NS = 16
L = 16
NW = NC * NS
NBUF = 5
DELTA = 2


@functools.lru_cache(maxsize=None)
def _make_gather(V, B):
    assert B % (CHUNK * NW) == 0
    n_per_w = B // NW
    n_chunks = n_per_w // CHUNK
    assert n_chunks % NBUF == 0

    mesh = plsc.VectorSubcoreMesh(core_axis_name="c", subcore_axis_name="s")

    @functools.partial(
        pl.kernel,
        mesh=mesh,
        out_type=jax.ShapeDtypeStruct((B, 2 * D), jnp.float32),
        compiler_params=pltpu.CompilerParams(use_tc_tiling_on_sc=False),
        scratch_types=(
            [pltpu.VMEM((n_chunks, CHUNK), jnp.int32)]
            + [pltpu.VMEM((CHUNK, D), jnp.float32) for _ in range(NBUF)]
            + [pltpu.VMEM((CHUNK, D), jnp.float32) for _ in range(NBUF)]
            + [pltpu.SemaphoreType.DMA for _ in range(2 * NBUF)]
        ),
    )
    def k(idx_hbm, table_hbm, out_hbm, idx_v, *scr):
        rows = scr[:NBUF]
        outr = scr[NBUF:2 * NBUF]
        gsem = scr[2 * NBUF:3 * NBUF]
        ssem = scr[3 * NBUF:]

        wid = lax.axis_index("s") * NC + lax.axis_index("c")
        base = wid * n_per_w
        pltpu.sync_copy(idx_hbm.at[pl.ds(wid * n_chunks, n_chunks)], idx_v)

        lane_consts = [jnp.full((L,), r, jnp.int32) for r in range(L)]

        def gather_start(b, j):
            pltpu.async_copy(table_hbm.at[idx_v.at[j]], rows[b], gsem[b])

        def gather_wait(b):
            pltpu.make_async_copy(
                out_hbm.at[pl.ds(0, CHUNK)], rows[b], gsem[b]).wait()

        def store_start(b, j):
            pltpu.async_copy(
                outr[b],
                out_hbm.at[pl.ds(base + j * CHUNK, CHUNK), pl.ds(0, D)],
                ssem[b])

        def store_drain(b):
            pltpu.make_async_copy(
                out_hbm.at[pl.ds(0, CHUNK)], outr[b], ssem[b]).wait()

        def mask_fix(b, j):
            def group_body(g, cc):
                iv = idx_v[j, pl.ds(g * L, L)]
                maskf = jnp.where(iv == 0, 0.0, 1.0).astype(jnp.float32)
                for r in range(L):
                    mf = maskf.at[lane_consts[r]].get(
                        mode="promise_in_bounds")
                    row = g * L + r
                    for c in range(D // L):
                        outr[b][row, pl.ds(c * L, L)] = (
                            rows[b][row, pl.ds(c * L, L)] * mf)
                return cc

            lax.fori_loop(0, CHUNK // L, group_body, 0)

        for c in range(DELTA):
            gather_start(c, c)

        def outer_body(o, carry):
            for b in range(NBUF):
                j = o * NBUF + b
                jn = j + DELTA
                bn = (b + DELTA) % NBUF

                @pl.when(jn < n_chunks)
                def _issue():
                    gather_start(bn, jn)

                gather_wait(b)

                @pl.when(j >= NBUF)
                def _drain():
                    store_drain(b)

                mask_fix(b, j)
                store_start(b, j)
            return carry

        lax.fori_loop(0, n_chunks // NBUF, outer_body, 0)

        for b in range(NBUF):
            store_drain(b)

    return k


def kernel(data, table):
    b0, b1 = data.shape
    B = b0 * b1
    idx = data.reshape(B // CHUNK, CHUNK)
    out = _make_gather(table.shape[0], B)(idx, table)
    return out[:, :D].reshape(b0, b1, D)


# speedup vs baseline: 1.0035x; 1.0035x over previous
import functools

import jax
import jax.numpy as jnp
from jax import lax
from jax.experimental import pallas as pl
from jax.experimental.pallas import tpu as pltpu
from jax.experimental.pallas import tpu_sc as plsc

D = 64
CHUNK = 128
NC = 2
NS = 16
L = 16
NW = NC * NS
NBUF = 5
DELTA = 3


@functools.lru_cache(maxsize=None)
def _make_gather(V, B):
    assert B % (CHUNK * NW) == 0
    n_per_w = B // NW
    n_chunks = n_per_w // CHUNK
    assert n_chunks % NBUF == 0

    mesh = plsc.VectorSubcoreMesh(core_axis_name="c", subcore_axis_name="s")

    @functools.partial(
        pl.kernel,
        mesh=mesh,
        out_type=jax.ShapeDtypeStruct((B, 2 * D), jnp.float32),
        compiler_params=pltpu.CompilerParams(use_tc_tiling_on_sc=False),
        scratch_types=(
            [pltpu.VMEM((n_chunks, CHUNK), jnp.int32)]
            + [pltpu.VMEM((CHUNK, D), jnp.float32) for _ in range(NBUF)]
            + [pltpu.VMEM((CHUNK, D), jnp.float32) for _ in range(NBUF)]
            + [pltpu.SemaphoreType.DMA for _ in range(2 * NBUF)]
        ),
    )
    def k(idx_hbm, table_hbm, out_hbm, idx_v, *scr):
        rows = scr[:NBUF]
        outr = scr[NBUF:2 * NBUF]
        gsem = scr[2 * NBUF:3 * NBUF]
        ssem = scr[3 * NBUF:]

        wid = lax.axis_index("s") * NC + lax.axis_index("c")
        base = wid * n_per_w
        pltpu.sync_copy(idx_hbm.at[pl.ds(wid * n_chunks, n_chunks)], idx_v)

        lane_consts = [jnp.full((L,), r, jnp.int32) for r in range(L)]

        def gather_start(b, j):
            pltpu.async_copy(table_hbm.at[idx_v.at[j]], rows[b], gsem[b])

        def gather_wait(b):
            pltpu.make_async_copy(
                out_hbm.at[pl.ds(0, CHUNK)], rows[b], gsem[b]).wait()

        def store_start(b, j):
            pltpu.async_copy(
                outr[b],
                out_hbm.at[pl.ds(base + j * CHUNK, CHUNK), pl.ds(0, D)],
                ssem[b])

        def store_drain(b):
            pltpu.make_async_copy(
                out_hbm.at[pl.ds(0, CHUNK)], outr[b], ssem[b]).wait()

        def mask_fix(b, j):
            def group_body(g, cc):
                iv = idx_v[j, pl.ds(g * L, L)]
                maskf = jnp.where(iv == 0, 0.0, 1.0).astype(jnp.float32)
                for r in range(L):
                    mf = maskf.at[lane_consts[r]].get(
                        mode="promise_in_bounds")
                    row = g * L + r
                    for c in range(D // L):
                        outr[b][row, pl.ds(c * L, L)] = (
                            rows[b][row, pl.ds(c * L, L)] * mf)
                return cc

            lax.fori_loop(0, CHUNK // L, group_body, 0)

        for c in range(DELTA):
            gather_start(c, c)

        def outer_body(o, carry):
            for b in range(NBUF):
                j = o * NBUF + b
                jn = j + DELTA
                bn = (b + DELTA) % NBUF

                @pl.when(jn < n_chunks)
                def _issue():
                    gather_start(bn, jn)

                gather_wait(b)

                @pl.when(j >= NBUF)
                def _drain():
                    store_drain(b)

                mask_fix(b, j)
                store_start(b, j)
            return carry

        lax.fori_loop(0, n_chunks // NBUF, outer_body, 0)

        for b in range(NBUF):
            store_drain(b)

    return k


def kernel(data, table):
    b0, b1 = data.shape
    B = b0 * b1
    idx = data.reshape(B // CHUNK, CHUNK)
    out = _make_gather(table.shape[0], B)(idx, table)
    return out[:, :D].reshape(b0, b1, D)
